# baseline (speedup 1.0000x reference)
import jax
import jax.numpy as jnp
from jax import lax
from jax.experimental import pallas as pl
from jax.experimental.pallas import tpu as pltpu
from jax.experimental.pallas import tpu_sc as plsc

B, C = 2, 64
HT_BINS = 184 * 60
IM_BINS = 128 * 128
N_VOTES = 262144
P = B * C

NC, NS, L = 2, 16, 16
NW = NC * NS
CPW = P // NW

CHUNK = 4096
NCHUNK = N_VOTES // CHUNK
MASK14 = (1 << 14) - 1


def _ht2im_body(tbl_hbm, pk_hbm, out_hbm,
                tv0, tv1, tv2, tv3, av0, av1, av2, av3,
                pk0, w0, pk1, w1, sem0, sem1):
    wid = lax.axis_index("s") * NC + lax.axis_index("c")
    tables = (tv0, tv1, tv2, tv3)
    accums = (av0, av1, av2, av3)

    for c in range(CPW):
        pltpu.async_copy(
            tbl_hbm.at[pl.ds((wid * CPW + c) * HT_BINS, HT_BINS)],
            tables[c], sem1)

    zv = jnp.zeros((L,), jnp.float32)
    for c in range(CPW):
        @plsc.parallel_loop(0, IM_BINS, step=L, unroll=8)
        def _zero(i, c=c):
            accums[c][pl.ds(i, L)] = zv

    for c in range(CPW):
        pltpu.make_async_copy(
            tbl_hbm.at[pl.ds(0, HT_BINS)], tables[c], sem1).wait()

    def start(g, bufs, sem):
        pkb, wb = bufs
        off = g * CHUNK
        pltpu.async_copy(pk_hbm.at[pl.ds(off, CHUNK)], pkb, sem)
        pltpu.async_copy(pk_hbm.at[pl.ds(N_VOTES + off, CHUNK)], wb, sem)

    def wait(bufs, sem):
        pkb, wb = bufs
        pltpu.make_async_copy(pk_hbm.at[pl.ds(0, CHUNK)], pkb, sem).wait()
        pltpu.make_async_copy(pk_hbm.at[pl.ds(0, CHUNK)], wb, sem).wait()

    def compute(bufs):
        pkb, wb = bufs

        @plsc.parallel_loop(0, CHUNK, step=L, unroll=8)
        def _steps(base):
            pk = pkb[pl.ds(base, L)]
            ht = pk & MASK14
            im = lax.shift_right_logical(pk, 14)
            w = plsc.bitcast(wb[pl.ds(base, L)], jnp.float32)
            for c in range(CPW):
                g = plsc.load_gather(tables[c], [ht])
                plsc.addupdate_scatter(accums[c], [im], g * w)

    bufs0 = (pk0, w0)
    bufs1 = (pk1, w1)

    start(0, bufs0, sem0)

    def outer(gg, _):
        g0 = gg * 2
        start(g0 + 1, bufs1, sem1)
        wait(bufs0, sem0)
        compute(bufs0)

        @pl.when(gg + 1 < NCHUNK // 2)
        def _():
            start(g0 + 2, bufs0, sem0)

        wait(bufs1, sem1)
        compute(bufs1)
        return 0

    lax.fori_loop(0, NCHUNK // 2, outer, 0)

    for c in range(CPW):
        pltpu.sync_copy(accums[c],
                        out_hbm.at[pl.ds((wid * CPW + c) * IM_BINS, IM_BINS)])


@jax.jit
def _ht2im(tbl, packed):
    mesh = plsc.VectorSubcoreMesh(
        core_axis_name="c", subcore_axis_name="s",
        num_cores=NC, num_subcores=NS)
    run = pl.kernel(
        _ht2im_body,
        out_type=jax.ShapeDtypeStruct((P * IM_BINS,), jnp.float32),
        mesh=mesh,
        compiler_params=pltpu.CompilerParams(needs_layout_passes=False),
        scratch_types=(
            [pltpu.VMEM((HT_BINS,), jnp.float32) for _ in range(CPW)]
            + [pltpu.VMEM((IM_BINS,), jnp.float32) for _ in range(CPW)]
            + [
                pltpu.VMEM((CHUNK,), jnp.int32),
                pltpu.VMEM((CHUNK,), jnp.int32),
                pltpu.VMEM((CHUNK,), jnp.int32),
                pltpu.VMEM((CHUNK,), jnp.int32),
                pltpu.SemaphoreType.DMA,
                pltpu.SemaphoreType.DMA,
            ]
        ),
    )
    return run(tbl, packed)


ROWS = N_VOTES // 128


def _pack_body(vm_ref, out_ref):
    m = vm_ref[:]
    k = lax.broadcasted_iota(jnp.int32, (384, 128), 0)
    j3 = 3 * lax.broadcasted_iota(jnp.int32, (384, 128), 1)
    dn = (((1,), (0,)), ((), ()))
    hp = lax.Precision.HIGHEST

    def sel(o):
        s = (k == j3 + o).astype(jnp.float32)
        return lax.dot_general(m, s, dn, precision=hp)

    ht = sel(0).astype(jnp.int32)
    im = sel(1).astype(jnp.int32)
    w = sel(2)
    out_ref[0] = (im << 14) | ht
    out_ref[1] = lax.bitcast_convert_type(w, jnp.int32)


@jax.jit
def _pack(vm2):
    return pl.pallas_call(
        _pack_body,
        out_shape=jax.ShapeDtypeStruct((2, ROWS, 128), jnp.int32),
    )(vm2)


def kernel(input, vote_mapping):
    b, c, hh, hw = input.shape
    tbl = input.reshape(b * c * hh * hw)
    vm2 = vote_mapping.reshape(ROWS, 384)
    packed = _pack(vm2).reshape(2 * N_VOTES)
    out = _ht2im(tbl, packed)
    return out.reshape(b, c, 128, 128)


# speedup vs baseline: 1.5361x; 1.5361x over previous
import jax
import jax.numpy as jnp
from jax import lax
from jax.experimental import pallas as pl
from jax.experimental.pallas import tpu as pltpu
from jax.experimental.pallas import tpu_sc as plsc

B, C = 2, 64
HT_BINS = 184 * 60
IM_BINS = 128 * 128
N_VOTES = 262144
P = B * C

NC, NS, L = 2, 16, 16
NW = NC * NS
CPW = P // NW

CHUNK = 4096
NCHUNK = N_VOTES // CHUNK
MASK14 = (1 << 14) - 1


def _ht2im_body(tbl_hbm, pk_hbm, out_hbm,
                tv0, tv1, tv2, tv3, av0, av1, av2, av3,
                pk0, w0, pk1, w1, sem0, sem1):
    wid = lax.axis_index("s") * NC + lax.axis_index("c")
    tables = (tv0, tv1, tv2, tv3)
    accums = (av0, av1, av2, av3)

    pltpu.async_copy(pk_hbm.at[pl.ds(0, CHUNK)], pk0, sem0)
    pltpu.async_copy(pk_hbm.at[pl.ds(N_VOTES, CHUNK)], w0, sem0)

    for c in range(CPW):
        pltpu.async_copy(
            tbl_hbm.at[pl.ds((wid * CPW + c) * HT_BINS, HT_BINS)],
            tables[c], sem1)

    zv = jnp.zeros((L,), jnp.float32)
    for c in range(CPW):
        @plsc.parallel_loop(0, IM_BINS, step=L, unroll=8)
        def _zero(i, c=c):
            accums[c][pl.ds(i, L)] = zv

    for c in range(CPW):
        pltpu.make_async_copy(
            tbl_hbm.at[pl.ds(0, HT_BINS)], tables[c], sem1).wait()

    def start(g, bufs, sem):
        pkb, wb = bufs
        off = g * CHUNK
        pltpu.async_copy(pk_hbm.at[pl.ds(off, CHUNK)], pkb, sem)
        pltpu.async_copy(pk_hbm.at[pl.ds(N_VOTES + off, CHUNK)], wb, sem)

    def wait(bufs, sem):
        pkb, wb = bufs
        pltpu.make_async_copy(pk_hbm.at[pl.ds(0, CHUNK)], pkb, sem).wait()
        pltpu.make_async_copy(pk_hbm.at[pl.ds(0, CHUNK)], wb, sem).wait()

    def compute(bufs):
        pkb, wb = bufs

        @plsc.parallel_loop(0, CHUNK, step=L, unroll=8)
        def _steps(base):
            pk = pkb[pl.ds(base, L)]
            ht = pk & MASK14
            im = lax.shift_right_logical(pk, 14)
            w = plsc.bitcast(wb[pl.ds(base, L)], jnp.float32)
            for c in range(CPW):
                g = plsc.load_gather(tables[c], [ht])
                plsc.addupdate_scatter(accums[c], [im], g * w)

    bufs0 = (pk0, w0)
    bufs1 = (pk1, w1)


    def outer(gg, _):
        g0 = gg * 2
        start(g0 + 1, bufs1, sem1)
        wait(bufs0, sem0)
        compute(bufs0)

        @pl.when(gg + 1 < NCHUNK // 2)
        def _():
            start(g0 + 2, bufs0, sem0)

        wait(bufs1, sem1)
        compute(bufs1)
        return 0

    lax.fori_loop(0, NCHUNK // 2, outer, 0)

    for c in range(CPW):
        pltpu.sync_copy(accums[c],
                        out_hbm.at[pl.ds((wid * CPW + c) * IM_BINS, IM_BINS)])


@jax.jit
def _ht2im(tbl, packed):
    mesh = plsc.VectorSubcoreMesh(
        core_axis_name="c", subcore_axis_name="s",
        num_cores=NC, num_subcores=NS)
    run = pl.kernel(
        _ht2im_body,
        out_type=jax.ShapeDtypeStruct((P * IM_BINS,), jnp.float32),
        mesh=mesh,
        compiler_params=pltpu.CompilerParams(needs_layout_passes=False),
        scratch_types=(
            [pltpu.VMEM((HT_BINS,), jnp.float32) for _ in range(CPW)]
            + [pltpu.VMEM((IM_BINS,), jnp.float32) for _ in range(CPW)]
            + [
                pltpu.VMEM((CHUNK,), jnp.int32),
                pltpu.VMEM((CHUNK,), jnp.int32),
                pltpu.VMEM((CHUNK,), jnp.int32),
                pltpu.VMEM((CHUNK,), jnp.int32),
                pltpu.SemaphoreType.DMA,
                pltpu.SemaphoreType.DMA,
            ]
        ),
    )
    return run(tbl, packed)


def kernel(input, vote_mapping):
    b, c, hh, hw = input.shape
    tbl = input.reshape(b * c * hh * hw)
    ht = vote_mapping[:, 0].astype(jnp.int32)
    im = vote_mapping[:, 1].astype(jnp.int32)
    htim = (im << 14) | ht
    wbits = lax.bitcast_convert_type(vote_mapping[:, 2], jnp.int32)
    packed = jnp.concatenate([htim, wbits])
    out = _ht2im(tbl, packed)
    return out.reshape(b, c, 128, 128)
